# baseline (speedup 1.0000x reference)
import functools

import jax
import jax.numpy as jnp
from jax import lax
from jax.experimental import pallas as pl
from jax.experimental.pallas import tpu as pltpu
from jax.experimental.pallas import tpu_sc as plsc

HIDDEN = 768
EPS = 1e-12

_INFO = plsc.get_sparse_core_info()
_NC = _INFO.num_cores
_NS = _INFO.num_subcores
_NW = _NC * _NS

_B, _S = 4, 2048
_TOKENS = _B * _S
_PIECES = ((0, 3), (3, 1))
_CH = 64


def _sc_gather(ids_flat, token_table, row0, rows):
    mesh = plsc.VectorSubcoreMesh(core_axis_name="c", subcore_axis_name="s")
    n_tokens = rows * _S
    tok_per_w = n_tokens // _NW
    flat0 = row0 * _S
    nch = tok_per_w // _CH

    @functools.partial(
        pl.kernel,
        mesh=mesh,
        out_type=jax.ShapeDtypeStruct((n_tokens, HIDDEN), jnp.float32),
        scratch_types=[
            pltpu.VMEM((tok_per_w,), jnp.int32),
            pltpu.VMEM((_CH, HIDDEN), jnp.float32),
            pltpu.VMEM((_CH, HIDDEN), jnp.float32),
            pltpu.SemaphoreType.DMA,
            pltpu.SemaphoreType.DMA,
            pltpu.SemaphoreType.DMA,
            pltpu.SemaphoreType.DMA,
        ],
    )
    def k(ids_ref, table_ref, out_ref, idx_v, buf0, buf1,
          sem0, sem1, wsem0, wsem1):
        wid = lax.axis_index("s") * _NC + lax.axis_index("c")
        base = wid * tok_per_w
        pltpu.sync_copy(ids_ref.at[pl.ds(flat0 + base, tok_per_w)], idx_v)
        bufs = (buf0, buf1)
        gsems = (sem0, sem1)
        wsems = (wsem0, wsem1)
        gcps = [None] * nch
        wcps = [None] * nch
        waited = [False] * nch
        for c in range(min(2, nch)):
            gcps[c] = pltpu.async_copy(
                table_ref.at[idx_v.at[pl.ds(c * _CH, _CH)]],
                bufs[c], gsems[c % 2])
        for c in range(nch):
            gcps[c].wait()
            wcps[c] = pltpu.async_copy(
                bufs[c % 2], out_ref.at[pl.ds(base + c * _CH, _CH)],
                wsems[c % 2])
            if c + 2 < nch:
                wcps[c].wait()
                waited[c] = True
                gcps[c + 2] = pltpu.async_copy(
                    table_ref.at[idx_v.at[pl.ds((c + 2) * _CH, _CH)]],
                    bufs[c % 2], gsems[c % 2])
        for c in range(nch):
            if not waited[c]:
                wcps[c].wait()

    return k(ids_flat, token_table)


def _tc_ln_body(*refs):
    g_ref, p_ref, gamma_ref, beta_ref = refs[:4]
    o_ref = refs[-1]
    x = g_ref[...] + p_ref[...]
    mean = jnp.mean(x, axis=-1, keepdims=True)
    xc = x - mean
    var = jnp.mean(xc * xc, axis=-1, keepdims=True)
    o_ref[...] = ((xc * lax.rsqrt(var + EPS)) * gamma_ref[...][None, :]
                  + beta_ref[...][None, :])


def _tc_ln_into(gathered, pos_table, gamma, beta, dst, row0, rows):
    in_specs = [
        pl.BlockSpec((_S, HIDDEN), lambda j: (j, 0)),
        pl.BlockSpec((_S, HIDDEN), lambda j: (0, 0)),
        pl.BlockSpec((HIDDEN,), lambda j: (0,)),
        pl.BlockSpec((HIDDEN,), lambda j: (0,)),
    ]
    args = [gathered, pos_table, gamma, beta]
    aliases = {}
    if dst is not None:
        in_specs.append(pl.BlockSpec(memory_space=pltpu.MemorySpace.HBM))
        args.append(dst)
        aliases = {4: 0}
    return pl.pallas_call(
        _tc_ln_body,
        grid=(rows,),
        in_specs=in_specs,
        out_specs=pl.BlockSpec((_S, HIDDEN), lambda j: (row0 + j, 0)),
        out_shape=jax.ShapeDtypeStruct((_TOKENS, HIDDEN), jnp.float32),
        input_output_aliases=aliases,
    )(*args)


def kernel(input_ids, token_table, pos_table, gamma, beta):
    B, S = input_ids.shape
    ids_flat = input_ids.reshape(-1)
    g = [_sc_gather(ids_flat, token_table, r0, rr) for r0, rr in _PIECES]
    dst = None
    for (r0, rr), gh in zip(_PIECES, g):
        dst = _tc_ln_into(gh, pos_table, gamma, beta, dst, r0, rr)
    return dst.reshape(B, S, HIDDEN)
